# baseline (speedup 1.0000x reference)
import numpy as np
import jax
import jax.numpy as jnp
from jax import lax
from jax.experimental import pallas as pl
from jax.experimental.pallas import tpu as pltpu
from jax.experimental.pallas import tpu_sc as plsc

_B, _C, _H, _W, _Q = 32, 1, 64, 256, 1024
_NC, _NS = 2, 16
_NW = _NC * _NS
_RPW = _Q // _NW

_CHUNKS = [4, 3, 4, 3, 4, 3, 4, 3, 4]
_NSLOT = len(_CHUNKS) + 1

_PERM = np.random.default_rng(1).permutation(_Q).astype(np.int32)


def _build_gidx() -> np.ndarray:
    g = np.zeros((_NW, _NSLOT, 16), np.int32)
    for w in range(_NW):
        off = 0
        for c, ck in enumerate(_CHUNKS):
            g[w, c, :ck] = _PERM[w * _RPW + off : w * _RPW + off + ck]
            off += ck
        g[w, _NSLOT - 1, 0] = _PERM[w]
    return g.reshape(-1)


_GIDX_NP = _build_gidx()


def _run_jobs(jobs):
    n = len(jobs)
    gh = [None] * n
    sh = [None] * n
    gh[0] = jobs[0][0]()
    for t in range(n):
        if t + 1 < n:
            if t - 1 >= 0:
                sh[t - 1].wait()
            gh[t + 1] = jobs[t + 1][0]()
        gh[t].wait()
        sh[t] = jobs[t][1]()
    if n >= 2:
        sh[n - 2].wait()
    sh[n - 1].wait()


def _sc_body(qk, keyb, gidx, dq_k, out_k, idxv, buf_a, buf_b, ga, gb, sa, sb):
    wid = lax.axis_index("s") * _NC + lax.axis_index("c")
    base = wid * _RPW
    bufs = (buf_a, buf_b)
    gsems = (ga, gb)
    ssems = (sa, sb)

    pltpu.sync_copy(gidx.at[pl.ds(wid * _NSLOT * 16, _NSLOT * 16)], idxv)

    def gather_job(t, slot, ck):
        buf, sem = bufs[t % 2], gsems[t % 2]
        return lambda: pltpu.async_copy(
            qk.at[idxv.at[pl.ds(slot * 16, ck)]], buf.at[pl.ds(0, ck)], sem)

    def linear_in_job(t, off, ck):
        buf, sem = bufs[t % 2], gsems[t % 2]
        return lambda: pltpu.async_copy(
            keyb.at[pl.ds(off, ck)], buf.at[pl.ds(0, ck)], sem)

    def scatter_job(t, dst, off, ck):
        buf, sem = bufs[t % 2], ssems[t % 2]
        return lambda: pltpu.async_copy(
            buf.at[pl.ds(0, ck)], dst.at[pl.ds(off, ck)], sem)

    def queue_jobs(linear_batch):
        jobs = []
        t = 0
        off = 0
        for c, ck in enumerate(_CHUNKS):
            gj = linear_in_job(t, off, ck) if linear_batch \
                else gather_job(t, c, ck)
            jobs.append((gj, scatter_job(t, out_k, base + off, ck)))
            t += 1
            off += ck
        jobs.append((gather_job(t, _NSLOT - 1, 1),
                     scatter_job(t, dq_k, wid, 1)))
        return jobs

    @pl.when(wid == 0)
    def _():
        _run_jobs(queue_jobs(True))

    @pl.when(wid != 0)
    def _():
        _run_jobs(queue_jobs(False))


def _sc_kernel(queue_k, key):
    f32 = jnp.float32
    mesh = plsc.VectorSubcoreMesh(core_axis_name="c", subcore_axis_name="s")
    sc_call = pl.kernel(
        _sc_body,
        mesh=mesh,
        out_type=[
            jax.ShapeDtypeStruct((_B, _C, _H, _W), f32),
            jax.ShapeDtypeStruct((_Q, _C, _H, _W), f32),
        ],
        scratch_types=[
            pltpu.VMEM((_NSLOT * 16,), jnp.int32),
            pltpu.VMEM((4, _C, _H, _W), f32),
            pltpu.VMEM((3, _C, _H, _W), f32),
            pltpu.SemaphoreType.DMA,
            pltpu.SemaphoreType.DMA,
            pltpu.SemaphoreType.DMA,
            pltpu.SemaphoreType.DMA,
        ],
    )
    dq_k, nk = sc_call(queue_k, key, jnp.asarray(_GIDX_NP))
    return dq_k, nk


_NLANE = 8


def _tc_body(perm_ref, *refs):
    i = pl.program_id(0)
    qq = refs[0:_NLANE]
    qry = refs[_NLANE]
    nq, dqq = refs[_NLANE + 1], refs[_NLANE + 2]
    for j in range(_NLANE):
        row = i * _NLANE + j

        @pl.when(row < _B)
        def _(j=j):
            dqq[pl.ds(j, 1)] = qq[j][...]
            nq[pl.ds(j, 1)] = qry[pl.ds(j, 1)]

        @pl.when(row >= _B)
        def _(j=j):
            nq[pl.ds(j, 1)] = qq[j][...]


def _tc_kernel(queue_q, query):
    blk = (1, _C, _H, _W)
    blk8 = (_NLANE, _C, _H, _W)
    nb_batch = _B // _NLANE
    f32 = jnp.float32

    def src_map(j):
        def m(i, perm_ref):
            return (perm_ref[i * _NLANE + j], 0, 0, 0)
        return m

    def batch_map(i, perm_ref):
        return (jnp.minimum(i, nb_batch - 1), 0, 0, 0)

    def out_map(i, perm_ref):
        return (i, 0, 0, 0)

    lanes = range(_NLANE)
    nq, dqq = pl.pallas_call(
        _tc_body,
        grid_spec=pltpu.PrefetchScalarGridSpec(
            num_scalar_prefetch=1,
            grid=(_Q // _NLANE,),
            in_specs=(
                [pl.BlockSpec(blk, src_map(j)) for j in lanes]
                + [pl.BlockSpec(blk8, batch_map)]
            ),
            out_specs=[
                pl.BlockSpec(blk8, out_map),
                pl.BlockSpec(blk8, batch_map),
            ],
        ),
        out_shape=[
            jax.ShapeDtypeStruct((_Q, _C, _H, _W), f32),
            jax.ShapeDtypeStruct((_B, _C, _H, _W), f32),
        ],
    )(jnp.asarray(_PERM), *([queue_q] * _NLANE + [query]))
    return nq, dqq


def kernel(queue_q, queue_k, query, key):
    dq_k, nk = _sc_kernel(queue_k, key)
    nq, dq_q = _tc_kernel(queue_q, query)
    return (dq_q, dq_k, nq, nk)


# speedup vs baseline: 1.1970x; 1.1970x over previous
import numpy as np
import jax
import jax.numpy as jnp
from jax import lax
from jax.experimental import pallas as pl
from jax.experimental.pallas import tpu as pltpu
from jax.experimental.pallas import tpu_sc as plsc

_B, _C, _H, _W, _Q = 32, 1, 64, 256, 1024
_NC, _NS = 2, 16
_NW = _NC * _NS
_RPW = _Q // _NW

_CHUNKS = [4, 3, 4, 3, 4, 3, 4, 3, 4]
_NSLOT = len(_CHUNKS) + 1

_PERM = np.random.default_rng(1).permutation(_Q).astype(np.int32)


def _build_gidx() -> np.ndarray:
    g = np.zeros((_NW, _NSLOT, 16), np.int32)
    for w in range(_NW):
        off = 0
        for c, ck in enumerate(_CHUNKS):
            g[w, c, :ck] = _PERM[w * _RPW + off : w * _RPW + off + ck]
            off += ck
        g[w, _NSLOT - 1, 0] = _PERM[w]
    return g.reshape(-1)


_GIDX_NP = _build_gidx()


def _run_jobs(jobs):
    n = len(jobs)
    gh = [None] * n
    sh = [None] * n
    gh[0] = jobs[0][0]()
    for t in range(n):
        if t + 1 < n:
            if t - 1 >= 0:
                sh[t - 1].wait()
            gh[t + 1] = jobs[t + 1][0]()
        gh[t].wait()
        sh[t] = jobs[t][1]()
    if n >= 2:
        sh[n - 2].wait()
    sh[n - 1].wait()


def _sc_body(qk, keyb, gidx, dq_k, out_k, idxv, buf_a, buf_b, ga, gb, sa, sb):
    wid = lax.axis_index("s") * _NC + lax.axis_index("c")
    base = wid * _RPW
    bufs = (buf_a, buf_b)
    gsems = (ga, gb)
    ssems = (sa, sb)

    pltpu.sync_copy(gidx.at[pl.ds(wid * _NSLOT * 16, _NSLOT * 16)], idxv)

    def gather_job(t, slot, ck):
        buf, sem = bufs[t % 2], gsems[t % 2]
        return lambda: pltpu.async_copy(
            qk.at[idxv.at[pl.ds(slot * 16, ck)]], buf.at[pl.ds(0, ck)], sem)

    def linear_in_job(t, off, ck):
        buf, sem = bufs[t % 2], gsems[t % 2]
        return lambda: pltpu.async_copy(
            keyb.at[pl.ds(off, ck)], buf.at[pl.ds(0, ck)], sem)

    def scatter_job(t, dst, off, ck):
        buf, sem = bufs[t % 2], ssems[t % 2]
        return lambda: pltpu.async_copy(
            buf.at[pl.ds(0, ck)], dst.at[pl.ds(off, ck)], sem)

    def queue_jobs(linear_batch):
        jobs = []
        t = 0
        off = 0
        for c, ck in enumerate(_CHUNKS):
            gj = linear_in_job(t, off, ck) if linear_batch \
                else gather_job(t, c, ck)
            jobs.append((gj, scatter_job(t, out_k, base + off, ck)))
            t += 1
            off += ck
        jobs.append((gather_job(t, _NSLOT - 1, 1),
                     scatter_job(t, dq_k, wid, 1)))
        return jobs

    @pl.when(wid == 0)
    def _():
        _run_jobs(queue_jobs(True))

    @pl.when(wid != 0)
    def _():
        _run_jobs(queue_jobs(False))


def _sc_kernel(queue_k, key):
    f32 = jnp.float32
    mesh = plsc.VectorSubcoreMesh(core_axis_name="c", subcore_axis_name="s")
    sc_call = pl.kernel(
        _sc_body,
        mesh=mesh,
        out_type=[
            jax.ShapeDtypeStruct((_B, _C, _H, _W), f32),
            jax.ShapeDtypeStruct((_Q, _C, _H, _W), f32),
        ],
        scratch_types=[
            pltpu.VMEM((_NSLOT * 16,), jnp.int32),
            pltpu.VMEM((4, _C, _H, _W), f32),
            pltpu.VMEM((3, _C, _H, _W), f32),
            pltpu.SemaphoreType.DMA,
            pltpu.SemaphoreType.DMA,
            pltpu.SemaphoreType.DMA,
            pltpu.SemaphoreType.DMA,
        ],
    )
    dq_k, nk = sc_call(queue_k, key, jnp.asarray(_GIDX_NP))
    return dq_k, nk


_NLANE = 16


def _tc_body(perm_ref, *refs):
    i = pl.program_id(0)
    qq = refs[0:_NLANE]
    qry = refs[_NLANE]
    nq, dqq = refs[_NLANE + 1], refs[_NLANE + 2]
    for j in range(_NLANE):
        row = i * _NLANE + j

        @pl.when(row < _B)
        def _(j=j):
            dqq[pl.ds(j, 1)] = qq[j][...]
            nq[pl.ds(j, 1)] = qry[pl.ds(j, 1)]

        @pl.when(row >= _B)
        def _(j=j):
            nq[pl.ds(j, 1)] = qq[j][...]


def _tc_kernel(queue_q, query):
    blk = (1, _C, _H, _W)
    blk8 = (_NLANE, _C, _H, _W)
    nb_batch = _B // _NLANE
    f32 = jnp.float32

    def src_map(j):
        def m(i, perm_ref):
            return (perm_ref[i * _NLANE + j], 0, 0, 0)
        return m

    def batch_map(i, perm_ref):
        return (jnp.minimum(i, nb_batch - 1), 0, 0, 0)

    def out_map(i, perm_ref):
        return (i, 0, 0, 0)

    lanes = range(_NLANE)
    nq, dqq = pl.pallas_call(
        _tc_body,
        grid_spec=pltpu.PrefetchScalarGridSpec(
            num_scalar_prefetch=1,
            grid=(_Q // _NLANE,),
            in_specs=(
                [pl.BlockSpec(blk, src_map(j)) for j in lanes]
                + [pl.BlockSpec(blk8, batch_map)]
            ),
            out_specs=[
                pl.BlockSpec(blk8, out_map),
                pl.BlockSpec(blk8, batch_map),
            ],
        ),
        out_shape=[
            jax.ShapeDtypeStruct((_Q, _C, _H, _W), f32),
            jax.ShapeDtypeStruct((_B, _C, _H, _W), f32),
        ],
    )(jnp.asarray(_PERM), *([queue_q] * _NLANE + [query]))
    return nq, dqq


def kernel(queue_q, queue_k, query, key):
    dq_k, nk = _sc_kernel(queue_k, key)
    nq, dq_q = _tc_kernel(queue_q, query)
    return (dq_q, dq_k, nq, nk)


# speedup vs baseline: 1.3111x; 1.0954x over previous
import numpy as np
import jax
import jax.numpy as jnp
from jax import lax
from jax.experimental import pallas as pl
from jax.experimental.pallas import tpu as pltpu
from jax.experimental.pallas import tpu_sc as plsc

_B, _C, _H, _W, _Q = 32, 1, 64, 256, 1024
_NC, _NS = 2, 16
_NW = _NC * _NS
_RPW = _Q // _NW

_CHUNKS = [4, 3, 4, 3, 4, 3, 4, 3, 4]
_NSLOT = len(_CHUNKS) + 1

_PERM = np.random.default_rng(1).permutation(_Q).astype(np.int32)


def _build_gidx() -> np.ndarray:
    g = np.zeros((_NW, _NSLOT, 16), np.int32)
    for w in range(_NW):
        off = 0
        for c, ck in enumerate(_CHUNKS):
            g[w, c, :ck] = _PERM[w * _RPW + off : w * _RPW + off + ck]
            off += ck
        g[w, _NSLOT - 1, 0] = _PERM[w]
    return g.reshape(-1)


_GIDX_NP = _build_gidx()


def _run_jobs(jobs):
    n = len(jobs)
    gh = [None] * n
    sh = [None] * n
    gh[0] = jobs[0][0]()
    for t in range(n):
        if t + 1 < n:
            if t - 1 >= 0:
                sh[t - 1].wait()
            gh[t + 1] = jobs[t + 1][0]()
        gh[t].wait()
        sh[t] = jobs[t][1]()
    if n >= 2:
        sh[n - 2].wait()
    sh[n - 1].wait()


def _sc_body(qk, keyb, gidx, dq_k, out_k, idxv, buf_a, buf_b, ga, gb, sa, sb):
    wid = lax.axis_index("s") * _NC + lax.axis_index("c")
    base = wid * _RPW
    bufs = (buf_a, buf_b)
    gsems = (ga, gb)
    ssems = (sa, sb)

    pltpu.sync_copy(gidx.at[pl.ds(wid * _NSLOT * 16, _NSLOT * 16)], idxv)

    def gather_job(t, slot, ck):
        buf, sem = bufs[t % 2], gsems[t % 2]
        return lambda: pltpu.async_copy(
            qk.at[idxv.at[pl.ds(slot * 16, ck)]], buf.at[pl.ds(0, ck)], sem)

    def linear_in_job(t, off, ck):
        buf, sem = bufs[t % 2], gsems[t % 2]
        return lambda: pltpu.async_copy(
            keyb.at[pl.ds(off, ck)], buf.at[pl.ds(0, ck)], sem)

    def scatter_job(t, dst, off, ck):
        buf, sem = bufs[t % 2], ssems[t % 2]
        return lambda: pltpu.async_copy(
            buf.at[pl.ds(0, ck)], dst.at[pl.ds(off, ck)], sem)

    def queue_jobs(linear_batch):
        jobs = []
        t = 0
        off = 0
        for c, ck in enumerate(_CHUNKS):
            gj = linear_in_job(t, off, ck) if linear_batch \
                else gather_job(t, c, ck)
            jobs.append((gj, scatter_job(t, out_k, base + off, ck)))
            t += 1
            off += ck
        jobs.append((gather_job(t, _NSLOT - 1, 1),
                     scatter_job(t, dq_k, wid, 1)))
        return jobs

    @pl.when(wid == 0)
    def _():
        _run_jobs(queue_jobs(True))

    @pl.when(wid != 0)
    def _():
        _run_jobs(queue_jobs(False))


def _sc_kernel(queue_k, key):
    f32 = jnp.float32
    mesh = plsc.VectorSubcoreMesh(core_axis_name="c", subcore_axis_name="s")
    sc_call = pl.kernel(
        _sc_body,
        mesh=mesh,
        out_type=[
            jax.ShapeDtypeStruct((_B, _C, _H, _W), f32),
            jax.ShapeDtypeStruct((_Q, _C, _H, _W), f32),
        ],
        scratch_types=[
            pltpu.VMEM((_NSLOT * 16,), jnp.int32),
            pltpu.VMEM((4, _C, _H, _W), f32),
            pltpu.VMEM((3, _C, _H, _W), f32),
            pltpu.SemaphoreType.DMA,
            pltpu.SemaphoreType.DMA,
            pltpu.SemaphoreType.DMA,
            pltpu.SemaphoreType.DMA,
        ],
    )
    dq_k, nk = sc_call(queue_k, key, jnp.asarray(_GIDX_NP))
    return dq_k, nk


_NLANE = 32


def _tc_body(perm_ref, *refs):
    i = pl.program_id(0)
    qq = refs[0:_NLANE]
    qry = refs[_NLANE]
    nq, dqq = refs[_NLANE + 1], refs[_NLANE + 2]
    for j in range(_NLANE):
        row = i * _NLANE + j

        @pl.when(row < _B)
        def _(j=j):
            dqq[pl.ds(j, 1)] = qq[j][...]
            nq[pl.ds(j, 1)] = qry[pl.ds(j, 1)]

        @pl.when(row >= _B)
        def _(j=j):
            nq[pl.ds(j, 1)] = qq[j][...]


def _tc_kernel(queue_q, query):
    blk = (1, _C, _H, _W)
    blk8 = (_NLANE, _C, _H, _W)
    nb_batch = _B // _NLANE
    f32 = jnp.float32

    def src_map(j):
        def m(i, perm_ref):
            return (perm_ref[i * _NLANE + j], 0, 0, 0)
        return m

    def batch_map(i, perm_ref):
        return (jnp.minimum(i, nb_batch - 1), 0, 0, 0)

    def out_map(i, perm_ref):
        return (i, 0, 0, 0)

    lanes = range(_NLANE)
    nq, dqq = pl.pallas_call(
        _tc_body,
        grid_spec=pltpu.PrefetchScalarGridSpec(
            num_scalar_prefetch=1,
            grid=(_Q // _NLANE,),
            in_specs=(
                [pl.BlockSpec(blk, src_map(j)) for j in lanes]
                + [pl.BlockSpec(blk8, batch_map)]
            ),
            out_specs=[
                pl.BlockSpec(blk8, out_map),
                pl.BlockSpec(blk8, batch_map),
            ],
        ),
        out_shape=[
            jax.ShapeDtypeStruct((_Q, _C, _H, _W), f32),
            jax.ShapeDtypeStruct((_B, _C, _H, _W), f32),
        ],
    )(jnp.asarray(_PERM), *([queue_q] * _NLANE + [query]))
    return nq, dqq


def kernel(queue_q, queue_k, query, key):
    dq_k, nk = _sc_kernel(queue_k, key)
    nq, dq_q = _tc_kernel(queue_q, query)
    return (dq_q, dq_k, nq, nk)


# speedup vs baseline: 1.3122x; 1.0008x over previous
import numpy as np
import jax
import jax.numpy as jnp
from jax import lax
from jax.experimental import pallas as pl
from jax.experimental.pallas import tpu as pltpu
from jax.experimental.pallas import tpu_sc as plsc

_B, _C, _H, _W, _Q = 32, 1, 64, 256, 1024
_NC, _NS = 2, 16
_NW = _NC * _NS
_RPW = _Q // _NW

_CHUNKS = [2] * 16
_NSLOT = len(_CHUNKS) + 1
_NBUF = 3

_PERM = np.random.default_rng(1).permutation(_Q).astype(np.int32)


def _build_gidx() -> np.ndarray:
    g = np.zeros((_NW, _NSLOT, 16), np.int32)
    for w in range(_NW):
        off = 0
        for c, ck in enumerate(_CHUNKS):
            g[w, c, :ck] = _PERM[w * _RPW + off : w * _RPW + off + ck]
            off += ck
        g[w, _NSLOT - 1, 0] = _PERM[w]
    return g.reshape(-1)


_GIDX_NP = _build_gidx()


def _run_jobs(jobs):
    n = len(jobs)
    gh = [None] * n
    sh = [None] * n
    for t in range(min(_NBUF - 1, n)):
        gh[t] = jobs[t][0]()
    for t in range(n):
        if t + _NBUF - 1 < n:
            if t - 1 >= 0:
                sh[t - 1].wait()
            gh[t + _NBUF - 1] = jobs[t + _NBUF - 1][0]()
        gh[t].wait()
        sh[t] = jobs[t][1]()
    for t in range(max(0, n - _NBUF), n):
        sh[t].wait()


def _sc_body(qk, keyb, gidx, dq_k, out_k, idxv,
             buf_a, buf_b, buf_c, ga, gb, gc, sa, sb, sc):
    wid = lax.axis_index("s") * _NC + lax.axis_index("c")
    base = wid * _RPW
    bufs = (buf_a, buf_b, buf_c)
    gsems = (ga, gb, gc)
    ssems = (sa, sb, sc)

    pltpu.sync_copy(gidx.at[pl.ds(wid * _NSLOT * 16, _NSLOT * 16)], idxv)

    def gather_job(t, slot, ck):
        buf, sem = bufs[t % _NBUF], gsems[t % _NBUF]
        return lambda: pltpu.async_copy(
            qk.at[idxv.at[pl.ds(slot * 16, ck)]], buf.at[pl.ds(0, ck)], sem)

    def linear_in_job(t, off, ck):
        buf, sem = bufs[t % _NBUF], gsems[t % _NBUF]
        return lambda: pltpu.async_copy(
            keyb.at[pl.ds(off, ck)], buf.at[pl.ds(0, ck)], sem)

    def scatter_job(t, dst, off, ck):
        buf, sem = bufs[t % _NBUF], ssems[t % _NBUF]
        return lambda: pltpu.async_copy(
            buf.at[pl.ds(0, ck)], dst.at[pl.ds(off, ck)], sem)

    def queue_jobs(linear_batch):
        jobs = []
        t = 0
        off = 0
        for c, ck in enumerate(_CHUNKS):
            gj = linear_in_job(t, off, ck) if linear_batch \
                else gather_job(t, c, ck)
            jobs.append((gj, scatter_job(t, out_k, base + off, ck)))
            t += 1
            off += ck
        jobs.append((gather_job(t, _NSLOT - 1, 1),
                     scatter_job(t, dq_k, wid, 1)))
        return jobs

    @pl.when(wid == 0)
    def _():
        _run_jobs(queue_jobs(True))

    @pl.when(wid != 0)
    def _():
        _run_jobs(queue_jobs(False))


def _sc_kernel(queue_k, key):
    f32 = jnp.float32
    mesh = plsc.VectorSubcoreMesh(core_axis_name="c", subcore_axis_name="s")
    sc_call = pl.kernel(
        _sc_body,
        mesh=mesh,
        out_type=[
            jax.ShapeDtypeStruct((_B, _C, _H, _W), f32),
            jax.ShapeDtypeStruct((_Q, _C, _H, _W), f32),
        ],
        scratch_types=[
            pltpu.VMEM((_NSLOT * 16,), jnp.int32),
            pltpu.VMEM((2, _C, _H, _W), f32),
            pltpu.VMEM((2, _C, _H, _W), f32),
            pltpu.VMEM((2, _C, _H, _W), f32),
            pltpu.SemaphoreType.DMA,
            pltpu.SemaphoreType.DMA,
            pltpu.SemaphoreType.DMA,
            pltpu.SemaphoreType.DMA,
            pltpu.SemaphoreType.DMA,
            pltpu.SemaphoreType.DMA,
        ],
    )
    dq_k, nk = sc_call(queue_k, key, jnp.asarray(_GIDX_NP))
    return dq_k, nk


_NLANE = 32


def _tc_body(perm_ref, *refs):
    i = pl.program_id(0)
    qq = refs[0:_NLANE]
    qry = refs[_NLANE]
    nq, dqq = refs[_NLANE + 1], refs[_NLANE + 2]
    for j in range(_NLANE):
        row = i * _NLANE + j

        @pl.when(row < _B)
        def _(j=j):
            dqq[pl.ds(j, 1)] = qq[j][...]
            nq[pl.ds(j, 1)] = qry[pl.ds(j, 1)]

        @pl.when(row >= _B)
        def _(j=j):
            nq[pl.ds(j, 1)] = qq[j][...]


def _tc_kernel(queue_q, query):
    blk = (1, _C, _H, _W)
    blk8 = (_NLANE, _C, _H, _W)
    nb_batch = _B // _NLANE
    f32 = jnp.float32

    def src_map(j):
        def m(i, perm_ref):
            return (perm_ref[i * _NLANE + j], 0, 0, 0)
        return m

    def batch_map(i, perm_ref):
        return (jnp.minimum(i, nb_batch - 1), 0, 0, 0)

    def out_map(i, perm_ref):
        return (i, 0, 0, 0)

    lanes = range(_NLANE)
    nq, dqq = pl.pallas_call(
        _tc_body,
        grid_spec=pltpu.PrefetchScalarGridSpec(
            num_scalar_prefetch=1,
            grid=(_Q // _NLANE,),
            in_specs=(
                [pl.BlockSpec(blk, src_map(j)) for j in lanes]
                + [pl.BlockSpec(blk8, batch_map)]
            ),
            out_specs=[
                pl.BlockSpec(blk8, out_map),
                pl.BlockSpec(blk8, batch_map),
            ],
        ),
        out_shape=[
            jax.ShapeDtypeStruct((_Q, _C, _H, _W), f32),
            jax.ShapeDtypeStruct((_B, _C, _H, _W), f32),
        ],
    )(jnp.asarray(_PERM), *([queue_q] * _NLANE + [query]))
    return nq, dqq


def kernel(queue_q, queue_k, query, key):
    dq_k, nk = _sc_kernel(queue_k, key)
    nq, dq_q = _tc_kernel(queue_q, query)
    return (dq_q, dq_k, nq, nk)
